# Initial kernel scaffold; baseline (speedup 1.0000x reference)
import jax
import jax.numpy as jnp
from jax.experimental import pallas as pl


def kernel(x, edge_index, edge_type, weight, bias):
    raise NotImplementedError("write your pallas kernel here")


# speedup vs baseline: 20.2329x; 20.2329x over previous
import functools

import jax
import jax.numpy as jnp
from jax import lax
from jax.experimental import pallas as pl
from jax.experimental.pallas import tpu as pltpu
from jax.experimental.pallas import tpu_sc as plsc

N = 10000
E = 320000
IN_FEAT = 128
OUT_FEAT = 128
NUM_RELS = 8

NUM_CORES = 2
NUM_SUBCORES = 16
NW = NUM_CORES * NUM_SUBCORES
EPW = E // NW
CHUNK = 80
NCHUNK = EPW // CHUNK
NPAD = 10240
ROWS_PER_TILE = NPAD // NUM_SUBCORES

BN = 1000
NB = N // BN
BNP = 1024
NBP = NPAD // BNP


def _h_body(x_ref, w_ref, b_ref, o_ref):
    o_ref[0] = (
        jnp.dot(x_ref[...], w_ref[0], preferred_element_type=jnp.float32)
        + b_ref[0]
    )


def _transform_nodes(x, weight, bias):
    return pl.pallas_call(
        _h_body,
        grid=(NUM_RELS, NB),
        in_specs=[
            pl.BlockSpec((BN, IN_FEAT), lambda r, i: (i, 0)),
            pl.BlockSpec((1, IN_FEAT, OUT_FEAT), lambda r, i: (r, 0, 0)),
            pl.BlockSpec((1, 1, OUT_FEAT), lambda r, i: (r, 0, 0)),
        ],
        out_specs=pl.BlockSpec((1, BN, OUT_FEAT), lambda r, i: (r, i, 0)),
        out_shape=jax.ShapeDtypeStruct((NUM_RELS, N, OUT_FEAT), jnp.float32),
    )(x, weight, bias.reshape(NUM_RELS, 1, OUT_FEAT))


def _idx_body(src_ref, et_ref, o_ref):
    o_ref[...] = et_ref[...] * N + src_ref[...]


def _gather_index(src, edge_type):
    src2 = src.reshape(E // 128, 128)
    et2 = edge_type.reshape(E // 128, 128)
    return pl.pallas_call(
        _idx_body,
        out_shape=jax.ShapeDtypeStruct((E // 128, 128), jnp.int32),
    )(src2, et2)


def _sc_body(table_hbm, gidx_hbm, dst_hbm, zeros_hbm, parts_hbm,
             idx_v, dsti_v, rows_v, acc_sh, sem):
    c = lax.axis_index("c")
    s = lax.axis_index("s")
    wid = c * NUM_SUBCORES + s
    pltpu.sync_copy(zeros_hbm, acc_sh.at[pl.ds(s * ROWS_PER_TILE, ROWS_PER_TILE)])
    pltpu.sync_copy(gidx_hbm.at[wid], idx_v)
    pltpu.sync_copy(dst_hbm.at[wid], dsti_v)
    plsc.subcore_barrier()

    def body(j, carry):
        pltpu.async_copy(table_hbm.at[idx_v.at[j]], rows_v, sem).wait()
        pltpu.sync_copy(rows_v, acc_sh.at[dsti_v.at[j]], add=True)
        return carry

    lax.fori_loop(0, NCHUNK, body, 0)
    plsc.subcore_barrier()
    pltpu.sync_copy(
        acc_sh.at[pl.ds(s * ROWS_PER_TILE, ROWS_PER_TILE)],
        parts_hbm.at[c, pl.ds(s * ROWS_PER_TILE, ROWS_PER_TILE)],
    )


_sc_gather_scatter = functools.partial(
    pl.kernel,
    out_type=jax.ShapeDtypeStruct((NUM_CORES, NPAD, OUT_FEAT), jnp.float32),
    mesh=plsc.VectorSubcoreMesh(
        core_axis_name="c", subcore_axis_name="s",
        num_cores=NUM_CORES, num_subcores=NUM_SUBCORES,
    ),
    scratch_types=[
        pltpu.VMEM((NCHUNK, CHUNK), jnp.int32),
        pltpu.VMEM((NCHUNK, CHUNK), jnp.int32),
        pltpu.VMEM((CHUNK, OUT_FEAT), jnp.float32),
        pltpu.VMEM_SHARED((NPAD, OUT_FEAT), jnp.float32),
        pltpu.SemaphoreType.DMA,
    ],
)(_sc_body)


def _add_body(p_ref, o_ref):
    o_ref[...] = p_ref[0] + p_ref[1]


def _merge_parts(parts):
    return pl.pallas_call(
        _add_body,
        grid=(NBP,),
        in_specs=[pl.BlockSpec((NUM_CORES, BNP, OUT_FEAT), lambda i: (0, i, 0))],
        out_specs=pl.BlockSpec((BNP, OUT_FEAT), lambda i: (i, 0)),
        out_shape=jax.ShapeDtypeStruct((NPAD, OUT_FEAT), jnp.float32),
    )(parts)


def kernel(x, edge_index, edge_type, weight, bias):
    src = edge_index[0]
    dst = edge_index[1]
    table = _transform_nodes(x, weight, bias).reshape(NUM_RELS * N, OUT_FEAT)
    gidx = _gather_index(src, edge_type).reshape(NW, NCHUNK, CHUNK)
    dst3 = dst.reshape(NW, NCHUNK, CHUNK)
    zeros = jnp.zeros((ROWS_PER_TILE, OUT_FEAT), jnp.float32)
    parts = _sc_gather_scatter(table, gidx, dst3, zeros)
    return _merge_parts(parts)[:N]


# speedup vs baseline: 28.2929x; 1.3984x over previous
import functools

import jax
import jax.numpy as jnp
from jax import lax
from jax.experimental import pallas as pl
from jax.experimental.pallas import tpu as pltpu
from jax.experimental.pallas import tpu_sc as plsc

N = 10000
E = 320000
IN_FEAT = 128
OUT_FEAT = 128
NUM_RELS = 8

NUM_CORES = 2
NUM_SUBCORES = 16
NW = NUM_CORES * NUM_SUBCORES
EPW = E // NW
CHUNK = 80
NCHUNK = EPW // CHUNK
NPAD = 10240
ROWS_PER_TILE = NPAD // NUM_SUBCORES

DST_BITS = 14

BN = 1000
NB = N // BN
BNP = 1024
NBP = NPAD // BNP


def _h_body(x_ref, w_ref, b_ref, o_ref):
    o_ref[0] = (
        jnp.dot(x_ref[...], w_ref[0], preferred_element_type=jnp.float32)
        + b_ref[0]
    )


def _transform_nodes(x, weight, bias):
    return pl.pallas_call(
        _h_body,
        grid=(NUM_RELS, NB),
        in_specs=[
            pl.BlockSpec((BN, IN_FEAT), lambda r, i: (i, 0)),
            pl.BlockSpec((1, IN_FEAT, OUT_FEAT), lambda r, i: (r, 0, 0)),
            pl.BlockSpec((1, 1, OUT_FEAT), lambda r, i: (r, 0, 0)),
        ],
        out_specs=pl.BlockSpec((1, BN, OUT_FEAT), lambda r, i: (r, i, 0)),
        out_shape=jax.ShapeDtypeStruct((NUM_RELS, N, OUT_FEAT), jnp.float32),
    )(x, weight, bias.reshape(NUM_RELS, 1, OUT_FEAT))


def _idx_body(src_ref, et_ref, dst_ref, o_ref):
    o_ref[...] = ((et_ref[...] * N + src_ref[...]) << DST_BITS) | dst_ref[...]


def _gather_index(src, edge_type, dst):
    src2 = src.reshape(E // 128, 128)
    et2 = edge_type.reshape(E // 128, 128)
    dst2 = dst.reshape(E // 128, 128)
    return pl.pallas_call(
        _idx_body,
        out_shape=jax.ShapeDtypeStruct((E // 128, 128), jnp.int32),
    )(src2, et2, dst2)


def _sc_body(table_hbm, packed_hbm, zeros_hbm, parts_hbm,
             packed_v, idx_a, dst_a, idx_b, dst_b, rows_a, rows_b, acc_sh,
             sem_a, sem_b):
    c = lax.axis_index("c")
    s = lax.axis_index("s")
    wid = c * NUM_SUBCORES + s
    pltpu.sync_copy(zeros_hbm, acc_sh.at[pl.ds(s * ROWS_PER_TILE, ROWS_PER_TILE)])
    pltpu.sync_copy(packed_hbm.at[wid], packed_v)
    plsc.subcore_barrier()

    def unpack(k, idx_buf, dst_buf):
        for i in range(CHUNK // 16):
            p = packed_v[k, pl.ds(i * 16, 16)]
            idx_buf[pl.ds(i * 16, 16)] = p >> DST_BITS
            dst_buf[pl.ds(i * 16, 16)] = p & ((1 << DST_BITS) - 1)

    def wait_for(buf, sem):
        pltpu.make_async_copy(table_hbm.at[pl.ds(0, CHUNK)], buf, sem).wait()

    unpack(0, idx_a, dst_a)
    pltpu.async_copy(table_hbm.at[idx_a], rows_a, sem_a)

    def body(j, carry):
        c0 = 2 * j
        unpack(c0 + 1, idx_b, dst_b)
        pltpu.async_copy(table_hbm.at[idx_b], rows_b, sem_b)
        wait_for(rows_a, sem_a)
        pltpu.sync_copy(rows_a, acc_sh.at[dst_a], add=True)
        unpack(c0 + 2, idx_a, dst_a)
        pltpu.async_copy(table_hbm.at[idx_a], rows_a, sem_a)
        wait_for(rows_b, sem_b)
        pltpu.sync_copy(rows_b, acc_sh.at[dst_b], add=True)
        return carry

    lax.fori_loop(0, (NCHUNK - 1) // 2, body, 0)
    wait_for(rows_a, sem_a)
    pltpu.sync_copy(rows_a, acc_sh.at[dst_a], add=True)
    plsc.subcore_barrier()
    pltpu.sync_copy(
        acc_sh.at[pl.ds(s * ROWS_PER_TILE, ROWS_PER_TILE)],
        parts_hbm.at[c, pl.ds(s * ROWS_PER_TILE, ROWS_PER_TILE)],
    )


_sc_gather_scatter = functools.partial(
    pl.kernel,
    out_type=jax.ShapeDtypeStruct((NUM_CORES, NPAD, OUT_FEAT), jnp.float32),
    mesh=plsc.VectorSubcoreMesh(
        core_axis_name="c", subcore_axis_name="s",
        num_cores=NUM_CORES, num_subcores=NUM_SUBCORES,
    ),
    scratch_types=[
        pltpu.VMEM((NCHUNK, CHUNK), jnp.int32),
        pltpu.VMEM((CHUNK,), jnp.int32),
        pltpu.VMEM((CHUNK,), jnp.int32),
        pltpu.VMEM((CHUNK,), jnp.int32),
        pltpu.VMEM((CHUNK,), jnp.int32),
        pltpu.VMEM((CHUNK, OUT_FEAT), jnp.float32),
        pltpu.VMEM((CHUNK, OUT_FEAT), jnp.float32),
        pltpu.VMEM_SHARED((NPAD, OUT_FEAT), jnp.float32),
        pltpu.SemaphoreType.DMA,
        pltpu.SemaphoreType.DMA,
    ],
)(_sc_body)


def _add_body(p_ref, o_ref):
    o_ref[...] = p_ref[0] + p_ref[1]


def _merge_parts(parts):
    return pl.pallas_call(
        _add_body,
        grid=(NBP,),
        in_specs=[pl.BlockSpec((NUM_CORES, BNP, OUT_FEAT), lambda i: (0, i, 0))],
        out_specs=pl.BlockSpec((BNP, OUT_FEAT), lambda i: (i, 0)),
        out_shape=jax.ShapeDtypeStruct((NPAD, OUT_FEAT), jnp.float32),
    )(parts)


def kernel(x, edge_index, edge_type, weight, bias):
    src = edge_index[0]
    dst = edge_index[1]
    table = _transform_nodes(x, weight, bias).reshape(NUM_RELS * N, OUT_FEAT)
    packed = _gather_index(src, edge_type, dst).reshape(NW, NCHUNK, CHUNK)
    zeros = jnp.zeros((ROWS_PER_TILE, OUT_FEAT), jnp.float32)
    parts = _sc_gather_scatter(table, packed, zeros)
    return _merge_parts(parts)[:N]


# speedup vs baseline: 28.7279x; 1.0154x over previous
import functools

import jax
import jax.numpy as jnp
from jax import lax
from jax.experimental import pallas as pl
from jax.experimental.pallas import tpu as pltpu
from jax.experimental.pallas import tpu_sc as plsc

N = 10000
E = 320000
IN_FEAT = 128
OUT_FEAT = 128
NUM_RELS = 8

NUM_CORES = 2
NUM_SUBCORES = 16
NW = NUM_CORES * NUM_SUBCORES
EPW = E // NW
CHUNK = 80
NCHUNK = EPW // CHUNK
NPAD = 10240
ROWS_PER_TILE = NPAD // NUM_SUBCORES

DST_BITS = 14

BN = 5000
NB = N // BN
BE = 12800
BM = 80


def _h_body(x_ref, w_ref, b_ref, o_ref):
    o_ref[0] = (
        jnp.dot(x_ref[...], w_ref[0], preferred_element_type=jnp.float32)
        + b_ref[0]
    )


def _transform_nodes(x, weight, bias):
    return pl.pallas_call(
        _h_body,
        grid=(NUM_RELS, NB),
        in_specs=[
            pl.BlockSpec((BN, IN_FEAT), lambda r, i: (i, 0)),
            pl.BlockSpec((1, IN_FEAT, OUT_FEAT), lambda r, i: (r, 0, 0)),
            pl.BlockSpec((1, 1, OUT_FEAT), lambda r, i: (r, 0, 0)),
        ],
        out_specs=pl.BlockSpec((1, BN, OUT_FEAT), lambda r, i: (r, i, 0)),
        out_shape=jax.ShapeDtypeStruct((NUM_RELS, N, OUT_FEAT), jnp.float32),
    )(x, weight, bias.reshape(NUM_RELS, 1, OUT_FEAT))


def _idx_body(ei_ref, et_ref, o_ref):
    o_ref[...] = ((et_ref[...] * N + ei_ref[0]) << DST_BITS) | ei_ref[1]


def _gather_index(edge_index, edge_type):
    return pl.pallas_call(
        _idx_body,
        out_shape=jax.ShapeDtypeStruct((E,), jnp.int32),
    )(edge_index, edge_type)


def _sc_body(table_hbm, packed_hbm, zeros_hbm, parts_hbm,
             packed_v, idx_a, dst_a, idx_b, dst_b, rows_a, rows_b, acc_sh,
             sem_a, sem_b):
    c = lax.axis_index("c")
    s = lax.axis_index("s")
    wid = c * NUM_SUBCORES + s
    pltpu.sync_copy(zeros_hbm, acc_sh.at[pl.ds(s * ROWS_PER_TILE, ROWS_PER_TILE)])
    pltpu.sync_copy(packed_hbm.at[pl.ds(wid * EPW, EPW)], packed_v)
    plsc.subcore_barrier()

    def unpack(k, idx_buf, dst_buf):
        for i in range(CHUNK // 16):
            p = packed_v[pl.ds(k * CHUNK + i * 16, 16)]
            idx_buf[pl.ds(i * 16, 16)] = p >> DST_BITS
            dst_buf[pl.ds(i * 16, 16)] = p & ((1 << DST_BITS) - 1)

    def wait_for(buf, sem):
        pltpu.make_async_copy(table_hbm.at[pl.ds(0, CHUNK)], buf, sem).wait()

    unpack(0, idx_a, dst_a)
    pltpu.async_copy(table_hbm.at[idx_a], rows_a, sem_a)

    def body(j, carry):
        c0 = 2 * j
        unpack(c0 + 1, idx_b, dst_b)
        pltpu.async_copy(table_hbm.at[idx_b], rows_b, sem_b)
        wait_for(rows_a, sem_a)
        pltpu.sync_copy(rows_a, acc_sh.at[dst_a], add=True)
        unpack(c0 + 2, idx_a, dst_a)
        pltpu.async_copy(table_hbm.at[idx_a], rows_a, sem_a)
        wait_for(rows_b, sem_b)
        pltpu.sync_copy(rows_b, acc_sh.at[dst_b], add=True)
        return carry

    lax.fori_loop(0, (NCHUNK - 1) // 2, body, 0)
    wait_for(rows_a, sem_a)
    pltpu.sync_copy(rows_a, acc_sh.at[dst_a], add=True)
    plsc.subcore_barrier()
    pltpu.sync_copy(
        acc_sh.at[pl.ds(s * ROWS_PER_TILE, ROWS_PER_TILE)],
        parts_hbm.at[c, pl.ds(s * ROWS_PER_TILE, ROWS_PER_TILE)],
    )


_sc_gather_scatter = functools.partial(
    pl.kernel,
    out_type=jax.ShapeDtypeStruct((NUM_CORES, NPAD, OUT_FEAT), jnp.float32),
    mesh=plsc.VectorSubcoreMesh(
        core_axis_name="c", subcore_axis_name="s",
        num_cores=NUM_CORES, num_subcores=NUM_SUBCORES,
    ),
    scratch_types=[
        pltpu.VMEM((EPW,), jnp.int32),
        pltpu.VMEM((CHUNK,), jnp.int32),
        pltpu.VMEM((CHUNK,), jnp.int32),
        pltpu.VMEM((CHUNK,), jnp.int32),
        pltpu.VMEM((CHUNK,), jnp.int32),
        pltpu.VMEM((CHUNK, OUT_FEAT), jnp.float32),
        pltpu.VMEM((CHUNK, OUT_FEAT), jnp.float32),
        pltpu.VMEM_SHARED((NPAD, OUT_FEAT), jnp.float32),
        pltpu.SemaphoreType.DMA,
        pltpu.SemaphoreType.DMA,
    ],
)(_sc_body)


def _add_body(p_ref, o_ref):
    o_ref[...] = p_ref[0] + p_ref[1]


def _merge_parts(parts):
    return pl.pallas_call(
        _add_body,
        grid=(N // BM,),
        in_specs=[pl.BlockSpec((NUM_CORES, BM, OUT_FEAT), lambda i: (0, i, 0))],
        out_specs=pl.BlockSpec((BM, OUT_FEAT), lambda i: (i, 0)),
        out_shape=jax.ShapeDtypeStruct((N, OUT_FEAT), jnp.float32),
    )(parts)


def kernel(x, edge_index, edge_type, weight, bias):
    table = _transform_nodes(x, weight, bias).reshape(NUM_RELS * N, OUT_FEAT)
    packed = _gather_index(edge_index, edge_type)
    zeros = jnp.zeros((ROWS_PER_TILE, OUT_FEAT), jnp.float32)
    parts = _sc_gather_scatter(table, packed, zeros)
    return _merge_parts(parts)


# speedup vs baseline: 41.4305x; 1.4422x over previous
import functools

import jax
import jax.numpy as jnp
from jax import lax
from jax.experimental import pallas as pl
from jax.experimental.pallas import tpu as pltpu
from jax.experimental.pallas import tpu_sc as plsc

N = 10000
E = 320000
IN_FEAT = 128
OUT_FEAT = 128
NUM_RELS = 8

NUM_CORES = 2
NUM_SUBCORES = 16
NW = NUM_CORES * NUM_SUBCORES
EPW = E // NW
CHUNK = 80
NCHUNK = EPW // CHUNK
NPAD = 10240
ROWS_PER_TILE = NPAD // NUM_SUBCORES

DST_BITS = 14

BN = 10000
NB = N // BN
BM = 2000


def _h_body(x_ref, w_ref, b_ref, o_ref):
    o_ref[0] = (
        jnp.dot(x_ref[...], w_ref[0], preferred_element_type=jnp.float32)
        + b_ref[0]
    )


def _transform_nodes(x, weight, bias):
    return pl.pallas_call(
        _h_body,
        grid=(NUM_RELS, NB),
        in_specs=[
            pl.BlockSpec((BN, IN_FEAT), lambda r, i: (i, 0)),
            pl.BlockSpec((1, IN_FEAT, OUT_FEAT), lambda r, i: (r, 0, 0)),
            pl.BlockSpec((1, 1, OUT_FEAT), lambda r, i: (r, 0, 0)),
        ],
        out_specs=pl.BlockSpec((1, BN, OUT_FEAT), lambda r, i: (r, i, 0)),
        out_shape=jax.ShapeDtypeStruct((NUM_RELS, N, OUT_FEAT), jnp.float32),
    )(x, weight, bias.reshape(NUM_RELS, 1, OUT_FEAT))


def _idx_body(ei_ref, et_ref, o_ref):
    o_ref[...] = ((et_ref[...] * N + ei_ref[0]) << DST_BITS) | ei_ref[1]


def _gather_index(edge_index, edge_type):
    return pl.pallas_call(
        _idx_body,
        out_shape=jax.ShapeDtypeStruct((E,), jnp.int32),
    )(edge_index, edge_type)


def _sc_body(table_hbm, packed_hbm, zeros_hbm, parts_hbm,
             packed_v, idx_a, dst_a, idx_b, dst_b, rows_a, rows_b, acc_sh,
             sem_a, sem_b):
    c = lax.axis_index("c")
    s = lax.axis_index("s")
    wid = c * NUM_SUBCORES + s
    pltpu.sync_copy(zeros_hbm, acc_sh.at[pl.ds(s * ROWS_PER_TILE, ROWS_PER_TILE)])
    pltpu.sync_copy(packed_hbm.at[pl.ds(wid * EPW, EPW)], packed_v)
    plsc.subcore_barrier()

    def unpack(k, idx_buf, dst_buf):
        for i in range(CHUNK // 16):
            p = packed_v[pl.ds(k * CHUNK + i * 16, 16)]
            idx_buf[pl.ds(i * 16, 16)] = p >> DST_BITS
            dst_buf[pl.ds(i * 16, 16)] = p & ((1 << DST_BITS) - 1)

    def wait_for(buf, sem):
        pltpu.make_async_copy(table_hbm.at[pl.ds(0, CHUNK)], buf, sem).wait()

    unpack(0, idx_a, dst_a)
    pltpu.async_copy(table_hbm.at[idx_a], rows_a, sem_a)

    def body(j, carry):
        c0 = 2 * j
        unpack(c0 + 1, idx_b, dst_b)
        pltpu.async_copy(table_hbm.at[idx_b], rows_b, sem_b)
        wait_for(rows_a, sem_a)
        pltpu.sync_copy(rows_a, acc_sh.at[dst_a], add=True)
        unpack(c0 + 2, idx_a, dst_a)
        pltpu.async_copy(table_hbm.at[idx_a], rows_a, sem_a)
        wait_for(rows_b, sem_b)
        pltpu.sync_copy(rows_b, acc_sh.at[dst_b], add=True)
        return carry

    lax.fori_loop(0, (NCHUNK - 1) // 2, body, 0)
    wait_for(rows_a, sem_a)
    pltpu.sync_copy(rows_a, acc_sh.at[dst_a], add=True)
    plsc.subcore_barrier()
    pltpu.sync_copy(
        acc_sh.at[pl.ds(s * ROWS_PER_TILE, ROWS_PER_TILE)],
        parts_hbm.at[c, pl.ds(s * ROWS_PER_TILE, ROWS_PER_TILE)],
    )


_sc_gather_scatter = functools.partial(
    pl.kernel,
    out_type=jax.ShapeDtypeStruct((NUM_CORES, NPAD, OUT_FEAT), jnp.float32),
    mesh=plsc.VectorSubcoreMesh(
        core_axis_name="c", subcore_axis_name="s",
        num_cores=NUM_CORES, num_subcores=NUM_SUBCORES,
    ),
    scratch_types=[
        pltpu.VMEM((EPW,), jnp.int32),
        pltpu.VMEM((CHUNK,), jnp.int32),
        pltpu.VMEM((CHUNK,), jnp.int32),
        pltpu.VMEM((CHUNK,), jnp.int32),
        pltpu.VMEM((CHUNK,), jnp.int32),
        pltpu.VMEM((CHUNK, OUT_FEAT), jnp.float32),
        pltpu.VMEM((CHUNK, OUT_FEAT), jnp.float32),
        pltpu.VMEM_SHARED((NPAD, OUT_FEAT), jnp.float32),
        pltpu.SemaphoreType.DMA,
        pltpu.SemaphoreType.DMA,
    ],
)(_sc_body)


def _add_body(p_ref, o_ref):
    o_ref[...] = p_ref[0] + p_ref[1]


def _merge_parts(parts):
    return pl.pallas_call(
        _add_body,
        grid=(N // BM,),
        in_specs=[pl.BlockSpec((NUM_CORES, BM, OUT_FEAT), lambda i: (0, i, 0))],
        out_specs=pl.BlockSpec((BM, OUT_FEAT), lambda i: (i, 0)),
        out_shape=jax.ShapeDtypeStruct((N, OUT_FEAT), jnp.float32),
    )(parts)


def kernel(x, edge_index, edge_type, weight, bias):
    table = _transform_nodes(x, weight, bias).reshape(NUM_RELS * N, OUT_FEAT)
    packed = _gather_index(edge_index, edge_type)
    zeros = jnp.zeros((ROWS_PER_TILE, OUT_FEAT), jnp.float32)
    parts = _sc_gather_scatter(table, packed, zeros)
    return _merge_parts(parts)


# speedup vs baseline: 41.4828x; 1.0013x over previous
import functools

import jax
import jax.numpy as jnp
from jax import lax
from jax.experimental import pallas as pl
from jax.experimental.pallas import tpu as pltpu
from jax.experimental.pallas import tpu_sc as plsc

N = 10000
E = 320000
IN_FEAT = 128
OUT_FEAT = 128
NUM_RELS = 8

NUM_CORES = 2
NUM_SUBCORES = 16
NW = NUM_CORES * NUM_SUBCORES
EPW = E // NW
CHUNK = 80
NCHUNK = EPW // CHUNK
NPAD = 10240
ROWS_PER_TILE = NPAD // NUM_SUBCORES

DST_BITS = 14

BN = 10000
NB = N // BN
BM = 2000


def _h_body(x_ref, w_ref, b_ref, o_ref):
    o_ref[0] = (
        jnp.dot(x_ref[...], w_ref[0], preferred_element_type=jnp.float32)
        + b_ref[0]
    )


def _transform_nodes(x, weight, bias):
    return pl.pallas_call(
        _h_body,
        grid=(NUM_RELS, NB),
        in_specs=[
            pl.BlockSpec((BN, IN_FEAT), lambda r, i: (i, 0)),
            pl.BlockSpec((1, IN_FEAT, OUT_FEAT), lambda r, i: (r, 0, 0)),
            pl.BlockSpec((1, 1, OUT_FEAT), lambda r, i: (r, 0, 0)),
        ],
        out_specs=pl.BlockSpec((1, BN, OUT_FEAT), lambda r, i: (r, i, 0)),
        out_shape=jax.ShapeDtypeStruct((NUM_RELS, N, OUT_FEAT), jnp.float32),
    )(x, weight, bias.reshape(NUM_RELS, 1, OUT_FEAT))


def _idx_body(ei_ref, et_ref, o_ref):
    o_ref[...] = ((et_ref[...] * N + ei_ref[0]) << DST_BITS) | ei_ref[1]


def _gather_index(edge_index, edge_type):
    return pl.pallas_call(
        _idx_body,
        out_shape=jax.ShapeDtypeStruct((E,), jnp.int32),
    )(edge_index, edge_type)


def _sc_body(table_hbm, packed_hbm, zeros_hbm, parts_hbm,
             packed_v, idx_a, dst_a, idx_b, dst_b, rows_a, rows_b, acc_sh,
             gsem_a, gsem_b, ssem_a, ssem_b):
    c = lax.axis_index("c")
    s = lax.axis_index("s")
    wid = c * NUM_SUBCORES + s
    pltpu.sync_copy(zeros_hbm, acc_sh.at[pl.ds(s * ROWS_PER_TILE, ROWS_PER_TILE)])
    pltpu.sync_copy(packed_hbm.at[pl.ds(wid * EPW, EPW)], packed_v)
    plsc.subcore_barrier()

    def unpack(k, idx_buf, dst_buf):
        for i in range(CHUNK // 16):
            p = packed_v[pl.ds(k * CHUNK + i * 16, 16)]
            idx_buf[pl.ds(i * 16, 16)] = p >> DST_BITS
            dst_buf[pl.ds(i * 16, 16)] = p & ((1 << DST_BITS) - 1)

    def wait_for(buf, sem):
        pltpu.make_async_copy(table_hbm.at[pl.ds(0, CHUNK)], buf, sem).wait()

    rows = (rows_a, rows_b)
    idxb = (idx_a, idx_b)
    dstb = (dst_a, dst_b)
    gsem = (gsem_a, gsem_b)
    ssem = (ssem_a, ssem_b)

    def body(j, carry):
        for t in (0, 1):
            c_fire = 2 * j + t

            @pl.when(c_fire < NCHUNK)
            def _():
                @pl.when(c_fire >= 2)
                def _():
                    wait_for(rows[t], ssem[t])

                unpack(c_fire, idxb[t], dstb[t])
                pltpu.async_copy(table_hbm.at[idxb[t]], rows[t], gsem[t])

            @pl.when(c_fire >= 1)
            def _():
                u = (t + 1) % 2
                wait_for(rows[u], gsem[u])
                pltpu.async_copy(rows[u], acc_sh.at[dstb[u]], ssem[u], add=True)

        return carry

    lax.fori_loop(0, (NCHUNK + 1) // 2, body, 0)
    wait_for(rows_b, ssem_b)
    wait_for(rows_a, ssem_a)
    plsc.subcore_barrier()
    pltpu.sync_copy(
        acc_sh.at[pl.ds(s * ROWS_PER_TILE, ROWS_PER_TILE)],
        parts_hbm.at[c, pl.ds(s * ROWS_PER_TILE, ROWS_PER_TILE)],
    )


_sc_gather_scatter = functools.partial(
    pl.kernel,
    out_type=jax.ShapeDtypeStruct((NUM_CORES, NPAD, OUT_FEAT), jnp.float32),
    mesh=plsc.VectorSubcoreMesh(
        core_axis_name="c", subcore_axis_name="s",
        num_cores=NUM_CORES, num_subcores=NUM_SUBCORES,
    ),
    scratch_types=[
        pltpu.VMEM((EPW,), jnp.int32),
        pltpu.VMEM((CHUNK,), jnp.int32),
        pltpu.VMEM((CHUNK,), jnp.int32),
        pltpu.VMEM((CHUNK,), jnp.int32),
        pltpu.VMEM((CHUNK,), jnp.int32),
        pltpu.VMEM((CHUNK, OUT_FEAT), jnp.float32),
        pltpu.VMEM((CHUNK, OUT_FEAT), jnp.float32),
        pltpu.VMEM_SHARED((NPAD, OUT_FEAT), jnp.float32),
        pltpu.SemaphoreType.DMA,
        pltpu.SemaphoreType.DMA,
        pltpu.SemaphoreType.DMA,
        pltpu.SemaphoreType.DMA,
    ],
)(_sc_body)


def _add_body(p_ref, o_ref):
    o_ref[...] = p_ref[0] + p_ref[1]


def _merge_parts(parts):
    return pl.pallas_call(
        _add_body,
        grid=(N // BM,),
        in_specs=[pl.BlockSpec((NUM_CORES, BM, OUT_FEAT), lambda i: (0, i, 0))],
        out_specs=pl.BlockSpec((BM, OUT_FEAT), lambda i: (i, 0)),
        out_shape=jax.ShapeDtypeStruct((N, OUT_FEAT), jnp.float32),
    )(parts)


def kernel(x, edge_index, edge_type, weight, bias):
    table = _transform_nodes(x, weight, bias).reshape(NUM_RELS * N, OUT_FEAT)
    packed = _gather_index(edge_index, edge_type)
    zeros = jnp.zeros((ROWS_PER_TILE, OUT_FEAT), jnp.float32)
    parts = _sc_gather_scatter(table, packed, zeros)
    return _merge_parts(parts)


# speedup vs baseline: 44.8203x; 1.0805x over previous
import functools

import jax
import jax.numpy as jnp
from jax import lax
from jax.experimental import pallas as pl
from jax.experimental.pallas import tpu as pltpu
from jax.experimental.pallas import tpu_sc as plsc

N = 10000
E = 320000
IN_FEAT = 128
OUT_FEAT = 128
NUM_RELS = 8

NUM_CORES = 2
NUM_SUBCORES = 16
NW = NUM_CORES * NUM_SUBCORES
EPW = E // NW
CHUNK = 128
NCHUNK = EPW // CHUNK
TAIL = EPW - NCHUNK * CHUNK
NPAD = 10240
ROWS_PER_TILE = NPAD // NUM_SUBCORES

DST_BITS = 14

BN = 10000
NB = N // BN
BM = 2000


def _h_body(x_ref, w_ref, b_ref, o_ref):
    o_ref[0] = (
        jnp.dot(x_ref[...], w_ref[0], preferred_element_type=jnp.float32)
        + b_ref[0]
    )


def _transform_nodes(x, weight, bias):
    return pl.pallas_call(
        _h_body,
        grid=(NUM_RELS, NB),
        in_specs=[
            pl.BlockSpec((BN, IN_FEAT), lambda r, i: (i, 0)),
            pl.BlockSpec((1, IN_FEAT, OUT_FEAT), lambda r, i: (r, 0, 0)),
            pl.BlockSpec((1, 1, OUT_FEAT), lambda r, i: (r, 0, 0)),
        ],
        out_specs=pl.BlockSpec((1, BN, OUT_FEAT), lambda r, i: (r, i, 0)),
        out_shape=jax.ShapeDtypeStruct((NUM_RELS, N, OUT_FEAT), jnp.float32),
    )(x, weight, bias.reshape(NUM_RELS, 1, OUT_FEAT))


def _idx_body(ei_ref, et_ref, o_ref):
    o_ref[...] = ((et_ref[...] * N + ei_ref[0]) << DST_BITS) | ei_ref[1]


def _gather_index(edge_index, edge_type):
    return pl.pallas_call(
        _idx_body,
        out_shape=jax.ShapeDtypeStruct((E,), jnp.int32),
    )(edge_index, edge_type)


def _sc_body(table_hbm, packed_hbm, zeros_hbm, parts_hbm,
             pk_a, pk_b, idx_a, dst_a, idx_b, dst_b, rows_a, rows_b,
             pk_t, idx_t, dst_t, acc_sh,
             gsem_a, gsem_b, ssem_a, ssem_b, psem_a, psem_b):
    c = lax.axis_index("c")
    s = lax.axis_index("s")
    wid = c * NUM_SUBCORES + s
    base = wid * EPW
    pltpu.sync_copy(zeros_hbm, acc_sh.at[pl.ds(s * ROWS_PER_TILE, ROWS_PER_TILE)])
    plsc.subcore_barrier()

    def unpack(pbuf, idx_buf, dst_buf, n):
        for i in range(n // 16):
            p = pbuf[pl.ds(i * 16, 16)]
            idx_buf[pl.ds(i * 16, 16)] = p >> DST_BITS
            dst_buf[pl.ds(i * 16, 16)] = p & ((1 << DST_BITS) - 1)

    def wait_rows(buf, sem):
        pltpu.make_async_copy(table_hbm.at[pl.ds(0, CHUNK)], buf, sem).wait()

    def wait_pk(buf, sem):
        pltpu.make_async_copy(packed_hbm.at[pl.ds(0, CHUNK)], buf, sem).wait()

    pk = (pk_a, pk_b)
    rows = (rows_a, rows_b)
    idxb = (idx_a, idx_b)
    dstb = (dst_a, dst_b)
    gsem = (gsem_a, gsem_b)
    ssem = (ssem_a, ssem_b)
    psem = (psem_a, psem_b)

    pltpu.async_copy(packed_hbm.at[pl.ds(base, CHUNK)], pk_a, psem_a)
    pltpu.async_copy(packed_hbm.at[pl.ds(base + CHUNK, CHUNK)], pk_b, psem_b)

    def body(j, carry):
        for t in (0, 1):
            c_fire = 2 * j + t

            @pl.when(c_fire < NCHUNK)
            def _():
                @pl.when(c_fire >= 2)
                def _():
                    wait_rows(rows[t], ssem[t])

                wait_pk(pk[t], psem[t])
                unpack(pk[t], idxb[t], dstb[t], CHUNK)
                pltpu.async_copy(table_hbm.at[idxb[t]], rows[t], gsem[t])

                @pl.when(c_fire + 2 < NCHUNK)
                def _():
                    pltpu.async_copy(
                        packed_hbm.at[pl.ds(base + (c_fire + 2) * CHUNK, CHUNK)],
                        pk[t], psem[t])

            @pl.when((c_fire >= 1) & (c_fire <= NCHUNK))
            def _():
                u = (t + 1) % 2
                wait_rows(rows[u], gsem[u])
                pltpu.async_copy(rows[u], acc_sh.at[dstb[u]], ssem[u], add=True)

        return carry

    lax.fori_loop(0, (NCHUNK + 2) // 2, body, 0)
    wait_rows(rows_a, ssem_a)
    wait_rows(rows_b, ssem_b)
    pltpu.sync_copy(packed_hbm.at[pl.ds(base + NCHUNK * CHUNK, TAIL)], pk_t)
    unpack(pk_t, idx_t, dst_t, TAIL)
    pltpu.async_copy(table_hbm.at[idx_t], rows_a.at[pl.ds(0, TAIL)], gsem_a).wait()
    pltpu.sync_copy(rows_a.at[pl.ds(0, TAIL)], acc_sh.at[dst_t], add=True)
    plsc.subcore_barrier()
    pltpu.sync_copy(
        acc_sh.at[pl.ds(s * ROWS_PER_TILE, ROWS_PER_TILE)],
        parts_hbm.at[c, pl.ds(s * ROWS_PER_TILE, ROWS_PER_TILE)],
    )


_sc_gather_scatter = functools.partial(
    pl.kernel,
    out_type=jax.ShapeDtypeStruct((NUM_CORES, NPAD, OUT_FEAT), jnp.float32),
    mesh=plsc.VectorSubcoreMesh(
        core_axis_name="c", subcore_axis_name="s",
        num_cores=NUM_CORES, num_subcores=NUM_SUBCORES,
    ),
    scratch_types=[
        pltpu.VMEM((CHUNK,), jnp.int32),
        pltpu.VMEM((CHUNK,), jnp.int32),
        pltpu.VMEM((CHUNK,), jnp.int32),
        pltpu.VMEM((CHUNK,), jnp.int32),
        pltpu.VMEM((CHUNK,), jnp.int32),
        pltpu.VMEM((CHUNK,), jnp.int32),
        pltpu.VMEM((CHUNK, OUT_FEAT), jnp.float32),
        pltpu.VMEM((CHUNK, OUT_FEAT), jnp.float32),
        pltpu.VMEM((TAIL,), jnp.int32),
        pltpu.VMEM((TAIL,), jnp.int32),
        pltpu.VMEM((TAIL,), jnp.int32),
        pltpu.VMEM_SHARED((NPAD, OUT_FEAT), jnp.float32),
        pltpu.SemaphoreType.DMA,
        pltpu.SemaphoreType.DMA,
        pltpu.SemaphoreType.DMA,
        pltpu.SemaphoreType.DMA,
        pltpu.SemaphoreType.DMA,
        pltpu.SemaphoreType.DMA,
    ],
)(_sc_body)


def _add_body(p_ref, o_ref):
    o_ref[...] = p_ref[0] + p_ref[1]


def _merge_parts(parts):
    return pl.pallas_call(
        _add_body,
        grid=(N // BM,),
        in_specs=[pl.BlockSpec((NUM_CORES, BM, OUT_FEAT), lambda i: (0, i, 0))],
        out_specs=pl.BlockSpec((BM, OUT_FEAT), lambda i: (i, 0)),
        out_shape=jax.ShapeDtypeStruct((N, OUT_FEAT), jnp.float32),
    )(parts)


def kernel(x, edge_index, edge_type, weight, bias):
    table = _transform_nodes(x, weight, bias).reshape(NUM_RELS * N, OUT_FEAT)
    packed = _gather_index(edge_index, edge_type)
    zeros = jnp.zeros((ROWS_PER_TILE, OUT_FEAT), jnp.float32)
    parts = _sc_gather_scatter(table, packed, zeros)
    return _merge_parts(parts)


# speedup vs baseline: 45.2851x; 1.0104x over previous
import functools

import jax
import jax.numpy as jnp
from jax import lax
from jax.experimental import pallas as pl
from jax.experimental.pallas import tpu as pltpu
from jax.experimental.pallas import tpu_sc as plsc

N = 10000
E = 320000
IN_FEAT = 128
OUT_FEAT = 128
NUM_RELS = 8

NUM_CORES = 2
NUM_SUBCORES = 16
NW = NUM_CORES * NUM_SUBCORES
EPW = E // NW
CHUNK = 128
NCHUNK = EPW // CHUNK
TAIL = EPW - NCHUNK * CHUNK
NPAD = 10240
ROWS_PER_TILE = NPAD // NUM_SUBCORES

DST_BITS = 14

BN = 10000
NB = N // BN
BM = 2000


def _h_body(x_ref, w_ref, b_ref, ei_ref, et_ref, o_ref, op_ref, oz_ref):
    o_ref[0] = (
        jnp.dot(x_ref[...], w_ref[0], preferred_element_type=jnp.float32)
        + b_ref[0]
    )

    @pl.when(pl.program_id(0) == 0)
    def _():
        op_ref[...] = ((et_ref[...] * N + ei_ref[0]) << DST_BITS) | ei_ref[1]
        oz_ref[...] = jnp.zeros((ROWS_PER_TILE, OUT_FEAT), jnp.float32)


def _transform_nodes(x, weight, bias, edge_index, edge_type):
    return pl.pallas_call(
        _h_body,
        grid=(NUM_RELS,),
        in_specs=[
            pl.BlockSpec((N, IN_FEAT), lambda r: (0, 0)),
            pl.BlockSpec((1, IN_FEAT, OUT_FEAT), lambda r: (r, 0, 0)),
            pl.BlockSpec((1, 1, OUT_FEAT), lambda r: (r, 0, 0)),
            pl.BlockSpec((2, E), lambda r: (0, 0)),
            pl.BlockSpec((E,), lambda r: (0,)),
        ],
        out_specs=[
            pl.BlockSpec((1, N, OUT_FEAT), lambda r: (r, 0, 0)),
            pl.BlockSpec((E,), lambda r: (0,)),
            pl.BlockSpec((ROWS_PER_TILE, OUT_FEAT), lambda r: (0, 0)),
        ],
        out_shape=[
            jax.ShapeDtypeStruct((NUM_RELS, N, OUT_FEAT), jnp.float32),
            jax.ShapeDtypeStruct((E,), jnp.int32),
            jax.ShapeDtypeStruct((ROWS_PER_TILE, OUT_FEAT), jnp.float32),
        ],
    )(x, weight, bias.reshape(NUM_RELS, 1, OUT_FEAT), edge_index, edge_type)


def _sc_body(table_hbm, packed_hbm, zeros_hbm, parts_hbm,
             pk_a, pk_b, idx_a, dst_a, idx_b, dst_b, rows_a, rows_b,
             pk_t, idx_t, dst_t, acc_sh,
             gsem_a, gsem_b, ssem_a, ssem_b, psem_a, psem_b):
    c = lax.axis_index("c")
    s = lax.axis_index("s")
    wid = c * NUM_SUBCORES + s
    base = wid * EPW
    pltpu.sync_copy(zeros_hbm, acc_sh.at[pl.ds(s * ROWS_PER_TILE, ROWS_PER_TILE)])
    plsc.subcore_barrier()

    def unpack(pbuf, idx_buf, dst_buf, n):
        for i in range(n // 16):
            p = pbuf[pl.ds(i * 16, 16)]
            idx_buf[pl.ds(i * 16, 16)] = p >> DST_BITS
            dst_buf[pl.ds(i * 16, 16)] = p & ((1 << DST_BITS) - 1)

    def wait_rows(buf, sem):
        pltpu.make_async_copy(table_hbm.at[pl.ds(0, CHUNK)], buf, sem).wait()

    def wait_pk(buf, sem):
        pltpu.make_async_copy(packed_hbm.at[pl.ds(0, CHUNK)], buf, sem).wait()

    pk = (pk_a, pk_b)
    rows = (rows_a, rows_b)
    idxb = (idx_a, idx_b)
    dstb = (dst_a, dst_b)
    gsem = (gsem_a, gsem_b)
    ssem = (ssem_a, ssem_b)
    psem = (psem_a, psem_b)

    pltpu.async_copy(packed_hbm.at[pl.ds(base, CHUNK)], pk_a, psem_a)
    pltpu.async_copy(packed_hbm.at[pl.ds(base + CHUNK, CHUNK)], pk_b, psem_b)

    def body(j, carry):
        for t in (0, 1):
            c_fire = 2 * j + t

            @pl.when(c_fire < NCHUNK)
            def _():
                @pl.when(c_fire >= 2)
                def _():
                    wait_rows(rows[t], ssem[t])

                wait_pk(pk[t], psem[t])
                unpack(pk[t], idxb[t], dstb[t], CHUNK)
                pltpu.async_copy(table_hbm.at[idxb[t]], rows[t], gsem[t])

                @pl.when(c_fire + 2 < NCHUNK)
                def _():
                    pltpu.async_copy(
                        packed_hbm.at[pl.ds(base + (c_fire + 2) * CHUNK, CHUNK)],
                        pk[t], psem[t])

            @pl.when((c_fire >= 1) & (c_fire <= NCHUNK))
            def _():
                u = (t + 1) % 2
                wait_rows(rows[u], gsem[u])
                pltpu.async_copy(rows[u], acc_sh.at[dstb[u]], ssem[u], add=True)

        return carry

    lax.fori_loop(0, (NCHUNK + 2) // 2, body, 0)
    wait_rows(rows_a, ssem_a)
    wait_rows(rows_b, ssem_b)
    pltpu.sync_copy(packed_hbm.at[pl.ds(base + NCHUNK * CHUNK, TAIL)], pk_t)
    unpack(pk_t, idx_t, dst_t, TAIL)
    pltpu.async_copy(table_hbm.at[idx_t], rows_a.at[pl.ds(0, TAIL)], gsem_a).wait()
    pltpu.sync_copy(rows_a.at[pl.ds(0, TAIL)], acc_sh.at[dst_t], add=True)
    plsc.subcore_barrier()
    pltpu.sync_copy(
        acc_sh.at[pl.ds(s * ROWS_PER_TILE, ROWS_PER_TILE)],
        parts_hbm.at[c, pl.ds(s * ROWS_PER_TILE, ROWS_PER_TILE)],
    )


_sc_gather_scatter = functools.partial(
    pl.kernel,
    out_type=jax.ShapeDtypeStruct((NUM_CORES, NPAD, OUT_FEAT), jnp.float32),
    mesh=plsc.VectorSubcoreMesh(
        core_axis_name="c", subcore_axis_name="s",
        num_cores=NUM_CORES, num_subcores=NUM_SUBCORES,
    ),
    scratch_types=[
        pltpu.VMEM((CHUNK,), jnp.int32),
        pltpu.VMEM((CHUNK,), jnp.int32),
        pltpu.VMEM((CHUNK,), jnp.int32),
        pltpu.VMEM((CHUNK,), jnp.int32),
        pltpu.VMEM((CHUNK,), jnp.int32),
        pltpu.VMEM((CHUNK,), jnp.int32),
        pltpu.VMEM((CHUNK, OUT_FEAT), jnp.float32),
        pltpu.VMEM((CHUNK, OUT_FEAT), jnp.float32),
        pltpu.VMEM((TAIL,), jnp.int32),
        pltpu.VMEM((TAIL,), jnp.int32),
        pltpu.VMEM((TAIL,), jnp.int32),
        pltpu.VMEM_SHARED((NPAD, OUT_FEAT), jnp.float32),
        pltpu.SemaphoreType.DMA,
        pltpu.SemaphoreType.DMA,
        pltpu.SemaphoreType.DMA,
        pltpu.SemaphoreType.DMA,
        pltpu.SemaphoreType.DMA,
        pltpu.SemaphoreType.DMA,
    ],
)(_sc_body)


def _add_body(p_ref, o_ref):
    o_ref[...] = p_ref[0] + p_ref[1]


def _merge_parts(parts):
    return pl.pallas_call(
        _add_body,
        grid=(N // BM,),
        in_specs=[pl.BlockSpec((NUM_CORES, BM, OUT_FEAT), lambda i: (0, i, 0))],
        out_specs=pl.BlockSpec((BM, OUT_FEAT), lambda i: (i, 0)),
        out_shape=jax.ShapeDtypeStruct((N, OUT_FEAT), jnp.float32),
    )(parts)


def kernel(x, edge_index, edge_type, weight, bias):
    h, packed, zeros = _transform_nodes(x, weight, bias, edge_index, edge_type)
    table = h.reshape(NUM_RELS * N, OUT_FEAT)
    parts = _sc_gather_scatter(table, packed, zeros)
    return _merge_parts(parts)


# speedup vs baseline: 49.4456x; 1.0919x over previous
import functools

import jax
import jax.numpy as jnp
from jax import lax
from jax.experimental import pallas as pl
from jax.experimental.pallas import tpu as pltpu
from jax.experimental.pallas import tpu_sc as plsc

N = 10000
E = 320000
IN_FEAT = 128
OUT_FEAT = 128
NUM_RELS = 8

NUM_CORES = 2
NUM_SUBCORES = 16
NW = NUM_CORES * NUM_SUBCORES
EPW = E // NW
CHUNK = 80
NCHUNK = EPW // CHUNK
TAIL = EPW - NCHUNK * CHUNK
NBUF = 3
NPAD = 10240
ROWS_PER_TILE = NPAD // NUM_SUBCORES

DST_BITS = 14

BN = 10000
NB = N // BN
BM = 2000


def _h_body(x_ref, w_ref, b_ref, ei_ref, et_ref, o_ref, op_ref, oz_ref):
    o_ref[0] = (
        jnp.dot(x_ref[...], w_ref[0], preferred_element_type=jnp.float32)
        + b_ref[0]
    )

    @pl.when(pl.program_id(0) == 0)
    def _():
        op_ref[...] = ((et_ref[...] * N + ei_ref[0]) << DST_BITS) | ei_ref[1]
        oz_ref[...] = jnp.zeros((ROWS_PER_TILE, OUT_FEAT), jnp.float32)


def _transform_nodes(x, weight, bias, edge_index, edge_type):
    return pl.pallas_call(
        _h_body,
        grid=(NUM_RELS,),
        in_specs=[
            pl.BlockSpec((N, IN_FEAT), lambda r: (0, 0)),
            pl.BlockSpec((1, IN_FEAT, OUT_FEAT), lambda r: (r, 0, 0)),
            pl.BlockSpec((1, 1, OUT_FEAT), lambda r: (r, 0, 0)),
            pl.BlockSpec((2, E), lambda r: (0, 0)),
            pl.BlockSpec((E,), lambda r: (0,)),
        ],
        out_specs=[
            pl.BlockSpec((1, N, OUT_FEAT), lambda r: (r, 0, 0)),
            pl.BlockSpec((E,), lambda r: (0,)),
            pl.BlockSpec((ROWS_PER_TILE, OUT_FEAT), lambda r: (0, 0)),
        ],
        out_shape=[
            jax.ShapeDtypeStruct((NUM_RELS, N, OUT_FEAT), jnp.float32),
            jax.ShapeDtypeStruct((E,), jnp.int32),
            jax.ShapeDtypeStruct((ROWS_PER_TILE, OUT_FEAT), jnp.float32),
        ],
    )(x, weight, bias.reshape(NUM_RELS, 1, OUT_FEAT), edge_index, edge_type)


def _sc_body(table_hbm, packed_hbm, zeros_hbm, parts_hbm,
             pk_a, pk_b, pk_c, idx_a, dst_a, idx_b, dst_b, idx_c, dst_c,
             rows_a, rows_b, rows_c, acc_sh,
             gsem_a, gsem_b, gsem_c, ssem_a, ssem_b, ssem_c,
             psem_a, psem_b, psem_c):
    c = lax.axis_index("c")
    s = lax.axis_index("s")
    wid = c * NUM_SUBCORES + s
    base = wid * EPW

    def unpack(pbuf, idx_buf, dst_buf, n):
        for i in range(n // 16):
            p = pbuf[pl.ds(i * 16, 16)]
            idx_buf[pl.ds(i * 16, 16)] = p >> DST_BITS
            dst_buf[pl.ds(i * 16, 16)] = p & ((1 << DST_BITS) - 1)

    def wait_rows(buf, sem):
        pltpu.make_async_copy(table_hbm.at[pl.ds(0, CHUNK)], buf, sem).wait()

    def wait_pk(buf, sem):
        pltpu.make_async_copy(packed_hbm.at[pl.ds(0, CHUNK)], buf, sem).wait()

    pk = (pk_a, pk_b, pk_c)
    rows = (rows_a, rows_b, rows_c)
    idxb = (idx_a, idx_b, idx_c)
    dstb = (dst_a, dst_b, dst_c)
    gsem = (gsem_a, gsem_b, gsem_c)
    ssem = (ssem_a, ssem_b, ssem_c)
    psem = (psem_a, psem_b, psem_c)

    for t in range(NBUF):
        pltpu.async_copy(packed_hbm.at[pl.ds(base + t * CHUNK, CHUNK)],
                         pk[t], psem[t])
    pltpu.sync_copy(zeros_hbm, acc_sh.at[pl.ds(s * ROWS_PER_TILE, ROWS_PER_TILE)])
    plsc.subcore_barrier()

    def body(j, carry):
        for t in range(NBUF):
            c_fire = NBUF * j + t

            @pl.when(c_fire < NCHUNK)
            def _():
                @pl.when(c_fire >= NBUF)
                def _():
                    wait_rows(rows[t], ssem[t])

                wait_pk(pk[t], psem[t])
                unpack(pk[t], idxb[t], dstb[t], CHUNK)
                pltpu.async_copy(table_hbm.at[idxb[t]], rows[t], gsem[t])

                @pl.when(c_fire + NBUF < NCHUNK)
                def _():
                    pltpu.async_copy(
                        packed_hbm.at[pl.ds(base + (c_fire + NBUF) * CHUNK, CHUNK)],
                        pk[t], psem[t])

            c_ret = c_fire - (NBUF - 1)

            @pl.when((c_ret >= 0) & (c_ret < NCHUNK))
            def _():
                u = (t + 1) % NBUF
                wait_rows(rows[u], gsem[u])
                pltpu.async_copy(rows[u], acc_sh.at[dstb[u]], ssem[u], add=True)

        return carry

    lax.fori_loop(0, (NCHUNK + 2 * (NBUF - 1) + NBUF - 1) // NBUF, body, 0)
    for t in range(NBUF):
        wait_rows(rows[t], ssem[t])
    plsc.subcore_barrier()
    pltpu.sync_copy(
        acc_sh.at[pl.ds(s * ROWS_PER_TILE, ROWS_PER_TILE)],
        parts_hbm.at[c, pl.ds(s * ROWS_PER_TILE, ROWS_PER_TILE)],
    )


_sc_gather_scatter = functools.partial(
    pl.kernel,
    out_type=jax.ShapeDtypeStruct((NUM_CORES, NPAD, OUT_FEAT), jnp.float32),
    mesh=plsc.VectorSubcoreMesh(
        core_axis_name="c", subcore_axis_name="s",
        num_cores=NUM_CORES, num_subcores=NUM_SUBCORES,
    ),
    scratch_types=(
        [pltpu.VMEM((CHUNK,), jnp.int32)] * 9
        + [pltpu.VMEM((CHUNK, OUT_FEAT), jnp.float32)] * 3
        + [pltpu.VMEM_SHARED((NPAD, OUT_FEAT), jnp.float32)]
        + [pltpu.SemaphoreType.DMA] * 9
    ),
)(_sc_body)


def _add_body(p_ref, o_ref):
    o_ref[...] = p_ref[0] + p_ref[1]


def _merge_parts(parts):
    return pl.pallas_call(
        _add_body,
        grid=(N // BM,),
        in_specs=[pl.BlockSpec((NUM_CORES, BM, OUT_FEAT), lambda i: (0, i, 0))],
        out_specs=pl.BlockSpec((BM, OUT_FEAT), lambda i: (i, 0)),
        out_shape=jax.ShapeDtypeStruct((N, OUT_FEAT), jnp.float32),
    )(parts)


def kernel(x, edge_index, edge_type, weight, bias):
    h, packed, zeros = _transform_nodes(x, weight, bias, edge_index, edge_type)
    table = h.reshape(NUM_RELS * N, OUT_FEAT)
    parts = _sc_gather_scatter(table, packed, zeros)
    return _merge_parts(parts)
